# baseline (speedup 1.0000x reference)
import functools

import jax
import jax.numpy as jnp
from jax import lax
from jax.experimental import pallas as pl
from jax.experimental.pallas import tpu as pltpu
from jax.experimental.pallas import tpu_sc as plsc

EPS = 1e-12


@functools.cache
def _make_sc_gather(V, D, B):
    info = plsc.get_sparse_core_info()
    NC, NS = info.num_cores, info.num_subcores
    NW = NC * NS
    assert B % NW == 0
    b_per_w = B // NW
    CH = min(128, b_per_w)
    assert b_per_w % CH == 0
    n_ch = b_per_w // CH
    mesh = plsc.VectorSubcoreMesh(core_axis_name="c", subcore_axis_name="s")

    @functools.partial(
        pl.kernel,
        mesh=mesh,
        out_type=jax.ShapeDtypeStruct((B, D), jnp.float32),
        scratch_types=[
            pltpu.VMEM((b_per_w,), jnp.int32),
            pltpu.VMEM((b_per_w, D), jnp.float32),
            pltpu.SemaphoreType.DMA,
        ],
    )
    def gather(idx_hbm, table_hbm, out_hbm, idx_v, rows_v, sem):
        wid = lax.axis_index("s") * NC + lax.axis_index("c")
        base = wid * b_per_w
        pltpu.sync_copy(idx_hbm.at[pl.ds(base, b_per_w)], idx_v)
        copies = [
            pltpu.async_copy(
                table_hbm.at[idx_v.at[pl.ds(j * CH, CH)]],
                rows_v.at[pl.ds(j * CH, CH)],
                sem,
            )
            for j in range(n_ch)
        ]
        for c in copies:
            c.wait()
        pltpu.sync_copy(rows_v, out_hbm.at[pl.ds(base, b_per_w)])

    return gather


def _tc_body(x_ref, g_ref, bt_ref, w_ref, b_ref, o_ref):
    x = x_ref[...]
    mean = jnp.mean(x, axis=1, keepdims=True)
    xc = x - mean
    var = jnp.mean(xc * xc, axis=1, keepdims=True)
    xn = xc * lax.rsqrt(var + EPS)
    xn = xn * g_ref[...][None, :] + bt_ref[...][None, :]
    o_ref[...] = (
        jnp.dot(xn, w_ref[...], preferred_element_type=jnp.float32)
        + b_ref[...][None, :]
    )


def _tc_body_acc(x_ref, g_ref, bt_ref, w_ref, b_ref, acc_ref, o_ref):
    del acc_ref
    _tc_body(x_ref, g_ref, bt_ref, w_ref, b_ref, o_ref)


@functools.cache
def _make_tc_proj_chunk(B, Bc, D, H, BT, c, first):
    steps = Bc // BT
    base = c * steps
    in_specs = [
        pl.BlockSpec((BT, D), lambda i: (i, 0)),
        pl.BlockSpec((D,), lambda i: (0,)),
        pl.BlockSpec((D,), lambda i: (0,)),
        pl.BlockSpec((D, H), lambda i: (0, 0)),
        pl.BlockSpec((H,), lambda i: (0,)),
    ]
    out_spec = pl.BlockSpec((BT, H), lambda i: (base + i, 0))
    out_shape = jax.ShapeDtypeStruct((B, H), jnp.float32)
    if first:
        return pl.pallas_call(
            _tc_body,
            grid=(steps,),
            in_specs=in_specs,
            out_specs=out_spec,
            out_shape=out_shape,
        )
    return pl.pallas_call(
        _tc_body_acc,
        grid=(steps,),
        in_specs=in_specs + [pl.BlockSpec(memory_space=pl.ANY)],
        out_specs=out_spec,
        out_shape=out_shape,
        input_output_aliases={5: 0},
    )


@jax.jit
def kernel(input_ids, table, gamma, beta, W, b):
    nb, seq = input_ids.shape
    V, D = table.shape
    H = W.shape[1]
    B = nb * seq
    C = 2
    BT = 1024
    Bc = B // C
    idx = input_ids.reshape(C, Bc).astype(jnp.int32)
    gather = _make_sc_gather(V, D, Bc)
    rows = [gather(idx[c], table) for c in range(C)]
    acc = _make_tc_proj_chunk(B, Bc, D, H, BT, 0, True)(
        rows[0], gamma, beta, W, b
    )
    for c in range(1, C):
        acc = _make_tc_proj_chunk(B, Bc, D, H, BT, c, False)(
            rows[c], gamma, beta, W, b, acc
        )
    return acc.reshape(nb, seq, H)


# speedup vs baseline: 1.0155x; 1.0155x over previous
import functools

import jax
import jax.numpy as jnp
from jax import lax
from jax.experimental import pallas as pl
from jax.experimental.pallas import tpu as pltpu
from jax.experimental.pallas import tpu_sc as plsc

EPS = 1e-12


@functools.cache
def _make_sc_gather(V, D, B):
    info = plsc.get_sparse_core_info()
    NC, NS = info.num_cores, info.num_subcores
    NW = NC * NS
    assert B % NW == 0
    b_per_w = B // NW
    CH = min(128, b_per_w)
    assert b_per_w % CH == 0
    n_ch = b_per_w // CH
    mesh = plsc.VectorSubcoreMesh(core_axis_name="c", subcore_axis_name="s")

    @functools.partial(
        pl.kernel,
        mesh=mesh,
        out_type=jax.ShapeDtypeStruct((B, D), jnp.float32),
        scratch_types=[
            pltpu.VMEM((b_per_w,), jnp.int32),
            pltpu.VMEM((b_per_w, D), jnp.float32),
            pltpu.SemaphoreType.DMA,
        ],
    )
    def gather(idx_hbm, table_hbm, out_hbm, idx_v, rows_v, sem):
        wid = lax.axis_index("s") * NC + lax.axis_index("c")
        base = wid * b_per_w
        pltpu.sync_copy(idx_hbm.at[pl.ds(base, b_per_w)], idx_v)
        copies = [
            pltpu.async_copy(
                table_hbm.at[idx_v.at[pl.ds(j * CH, CH)]],
                rows_v.at[pl.ds(j * CH, CH)],
                sem,
            )
            for j in range(n_ch)
        ]
        for c in copies:
            c.wait()
        pltpu.sync_copy(rows_v, out_hbm.at[pl.ds(base, b_per_w)])

    return gather


def _tc_body(x_ref, g_ref, bt_ref, w_ref, b_ref, o_ref):
    x = x_ref[...]
    mean = jnp.mean(x, axis=1, keepdims=True)
    xc = x - mean
    var = jnp.mean(xc * xc, axis=1, keepdims=True)
    xn = xc * lax.rsqrt(var + EPS)
    xn = xn * g_ref[...][None, :] + bt_ref[...][None, :]
    o_ref[...] = (
        jnp.dot(xn, w_ref[...], preferred_element_type=jnp.float32)
        + b_ref[...][None, :]
    )


def _tc_body_acc(x_ref, g_ref, bt_ref, w_ref, b_ref, acc_ref, o_ref):
    del acc_ref
    _tc_body(x_ref, g_ref, bt_ref, w_ref, b_ref, o_ref)


@functools.cache
def _make_tc_proj_chunk(B, Bc, D, H, BT, c, first):
    steps = Bc // BT
    base = c * steps
    in_specs = [
        pl.BlockSpec((BT, D), lambda i: (i, 0)),
        pl.BlockSpec((D,), lambda i: (0,)),
        pl.BlockSpec((D,), lambda i: (0,)),
        pl.BlockSpec((D, H), lambda i: (0, 0)),
        pl.BlockSpec((H,), lambda i: (0,)),
    ]
    out_spec = pl.BlockSpec((BT, H), lambda i: (base + i, 0))
    out_shape = jax.ShapeDtypeStruct((B, H), jnp.float32)
    if first:
        return pl.pallas_call(
            _tc_body,
            grid=(steps,),
            in_specs=in_specs,
            out_specs=out_spec,
            out_shape=out_shape,
        )
    return pl.pallas_call(
        _tc_body_acc,
        grid=(steps,),
        in_specs=in_specs + [pl.BlockSpec(memory_space=pl.ANY)],
        out_specs=out_spec,
        out_shape=out_shape,
        input_output_aliases={5: 0},
    )


@jax.jit
def kernel(input_ids, table, gamma, beta, W, b):
    nb, seq = input_ids.shape
    V, D = table.shape
    H = W.shape[1]
    B = nb * seq
    C = 2
    BT = 2048
    Bc = B // C
    idx = input_ids.reshape(C, Bc).astype(jnp.int32)
    gather = _make_sc_gather(V, D, Bc)
    rows = [gather(idx[c], table) for c in range(C)]
    acc = _make_tc_proj_chunk(B, Bc, D, H, BT, 0, True)(
        rows[0], gamma, beta, W, b
    )
    for c in range(1, C):
        acc = _make_tc_proj_chunk(B, Bc, D, H, BT, c, False)(
            rows[c], gamma, beta, W, b, acc
        )
    return acc.reshape(nb, seq, H)


# speedup vs baseline: 1.0970x; 1.0802x over previous
import functools

import jax
import jax.numpy as jnp
from jax import lax
from jax.experimental import pallas as pl
from jax.experimental.pallas import tpu as pltpu
from jax.experimental.pallas import tpu_sc as plsc

EPS = 1e-12


@functools.cache
def _make_sc_gather(V, D, B):
    info = plsc.get_sparse_core_info()
    NC, NS = info.num_cores, info.num_subcores
    NW = NC * NS
    assert B % NW == 0
    b_per_w = B // NW
    CH = 64
    assert b_per_w % CH == 0
    n_ch = b_per_w // CH
    mesh = plsc.VectorSubcoreMesh(core_axis_name="c", subcore_axis_name="s")

    @functools.partial(
        pl.kernel,
        mesh=mesh,
        out_type=jax.ShapeDtypeStruct((B, D), jnp.float32),
        scratch_types=[
            pltpu.VMEM((b_per_w,), jnp.int32),
            pltpu.VMEM((b_per_w, D), jnp.float32),
        ]
        + [pltpu.SemaphoreType.DMA] * (2 * n_ch),
    )
    def gather(idx_hbm, table_hbm, out_hbm, idx_v, rows_v, *sems):
        wid = lax.axis_index("s") * NC + lax.axis_index("c")
        base = wid * b_per_w
        pltpu.sync_copy(idx_hbm.at[pl.ds(base, b_per_w)], idx_v)
        gathers = [
            pltpu.async_copy(
                table_hbm.at[idx_v.at[pl.ds(j * CH, CH)]],
                rows_v.at[pl.ds(j * CH, CH)],
                sems[j],
            )
            for j in range(n_ch)
        ]
        writes = []
        for j in range(n_ch):
            gathers[j].wait()
            writes.append(
                pltpu.async_copy(
                    rows_v.at[pl.ds(j * CH, CH)],
                    out_hbm.at[pl.ds(base + j * CH, CH)],
                    sems[n_ch + j],
                )
            )
        for w in writes:
            w.wait()

    return gather


def _tc_body(x_ref, g_ref, bt_ref, w_ref, b_ref, o_ref):
    x = x_ref[...]
    mean = jnp.mean(x, axis=1, keepdims=True)
    xc = x - mean
    var = jnp.mean(xc * xc, axis=1, keepdims=True)
    xn = xc * lax.rsqrt(var + EPS)
    xn = xn * g_ref[...][None, :] + bt_ref[...][None, :]
    o_ref[...] = (
        jnp.dot(xn, w_ref[...], preferred_element_type=jnp.float32)
        + b_ref[...][None, :]
    )


@functools.cache
def _make_tc_proj(B, D, H, BT):
    return pl.pallas_call(
        _tc_body,
        grid=(B // BT,),
        in_specs=[
            pl.BlockSpec((BT, D), lambda i: (i, 0)),
            pl.BlockSpec((D,), lambda i: (0,)),
            pl.BlockSpec((D,), lambda i: (0,)),
            pl.BlockSpec((D, H), lambda i: (0, 0)),
            pl.BlockSpec((H,), lambda i: (0,)),
        ],
        out_specs=pl.BlockSpec((BT, H), lambda i: (i, 0)),
        out_shape=jax.ShapeDtypeStruct((B, H), jnp.float32),
    )


@jax.jit
def kernel(input_ids, table, gamma, beta, W, b):
    nb, seq = input_ids.shape
    V, D = table.shape
    H = W.shape[1]
    B = nb * seq
    idx = input_ids.reshape(B).astype(jnp.int32)
    rows = _make_sc_gather(V, D, B)(idx, table)
    out = _make_tc_proj(B, D, H, 2048)(rows, gamma, beta, W, b)
    return out.reshape(nb, seq, H)


# speedup vs baseline: 1.1008x; 1.0035x over previous
import functools

import jax
import jax.numpy as jnp
from jax import lax
from jax.experimental import pallas as pl
from jax.experimental.pallas import tpu as pltpu
from jax.experimental.pallas import tpu_sc as plsc

EPS = 1e-12


@functools.cache
def _make_sc_gather(V, D, B):
    info = plsc.get_sparse_core_info()
    NC, NS = info.num_cores, info.num_subcores
    NW = NC * NS
    assert B % NW == 0
    b_per_w = B // NW
    CH = 64
    assert b_per_w % CH == 0
    n_ch = b_per_w // CH
    mesh = plsc.VectorSubcoreMesh(core_axis_name="c", subcore_axis_name="s")

    @functools.partial(
        pl.kernel,
        mesh=mesh,
        out_type=jax.ShapeDtypeStruct((B, D), jnp.float32),
        scratch_types=[
            pltpu.VMEM((b_per_w,), jnp.int32),
            pltpu.VMEM((b_per_w, D), jnp.float32),
        ]
        + [pltpu.SemaphoreType.DMA] * (2 * n_ch),
    )
    def gather(idx_hbm, table_hbm, out_hbm, idx_v, rows_v, *sems):
        wid = lax.axis_index("s") * NC + lax.axis_index("c")
        base = wid * b_per_w
        pltpu.sync_copy(idx_hbm.at[pl.ds(base, b_per_w)], idx_v)
        gathers = [
            pltpu.async_copy(
                table_hbm.at[idx_v.at[pl.ds(j * CH, CH)]],
                rows_v.at[pl.ds(j * CH, CH)],
                sems[j],
            )
            for j in range(n_ch)
        ]
        writes = []
        for j in range(n_ch):
            gathers[j].wait()
            writes.append(
                pltpu.async_copy(
                    rows_v.at[pl.ds(j * CH, CH)],
                    out_hbm.at[pl.ds(base + j * CH, CH)],
                    sems[n_ch + j],
                )
            )
        for w in writes:
            w.wait()

    return gather


@functools.cache
def _make_tc_proj(B, D, H, BT, NB):
    S = B // BT

    def body(x_hbm, g_ref, bt_ref, w_ref, b_ref, o_hbm, xb, ob, sx, so):
        def xcopy(s):
            return pltpu.make_async_copy(
                x_hbm.at[pl.ds(s * BT, BT)], xb.at[s % NB], sx.at[s % NB]
            )

        def ocopy(s):
            return pltpu.make_async_copy(
                ob.at[s % NB], o_hbm.at[pl.ds(s * BT, BT)], so.at[s % NB]
            )

        for s in range(min(NB, S)):
            xcopy(s).start()
        gam = g_ref[...][None, :]
        bet = bt_ref[...][None, :]
        w = w_ref[...]
        bias = b_ref[...][None, :]
        for s in range(S):
            xcopy(s).wait()
            x = xb[s % NB]
            mean = jnp.mean(x, axis=1, keepdims=True)
            xc = x - mean
            var = jnp.mean(xc * xc, axis=1, keepdims=True)
            xn = xc * lax.rsqrt(var + EPS) * gam + bet
            if s >= NB:
                ocopy(s - NB).wait()
            ob[s % NB] = (
                jnp.dot(xn, w, preferred_element_type=jnp.float32) + bias
            )
            ocopy(s).start()
            if s + NB < S:
                xcopy(s + NB).start()
        for s in range(max(S - NB, 0), S):
            ocopy(s).wait()

    return pl.pallas_call(
        body,
        in_specs=[
            pl.BlockSpec(memory_space=pl.ANY),
            pl.BlockSpec(memory_space=pltpu.MemorySpace.VMEM),
            pl.BlockSpec(memory_space=pltpu.MemorySpace.VMEM),
            pl.BlockSpec(memory_space=pltpu.MemorySpace.VMEM),
            pl.BlockSpec(memory_space=pltpu.MemorySpace.VMEM),
        ],
        out_specs=pl.BlockSpec(memory_space=pl.ANY),
        out_shape=jax.ShapeDtypeStruct((B, H), jnp.float32),
        scratch_shapes=[
            pltpu.VMEM((NB, BT, D), jnp.float32),
            pltpu.VMEM((NB, BT, H), jnp.float32),
            pltpu.SemaphoreType.DMA((NB,)),
            pltpu.SemaphoreType.DMA((NB,)),
        ],
    )


@jax.jit
def kernel(input_ids, table, gamma, beta, W, b):
    nb, seq = input_ids.shape
    V, D = table.shape
    H = W.shape[1]
    B = nb * seq
    idx = input_ids.reshape(B).astype(jnp.int32)
    rows = _make_sc_gather(V, D, B)(idx, table)
    out = _make_tc_proj(B, D, H, 512, 4)(rows, gamma, beta, W, b)
    return out.reshape(nb, seq, H)


# speedup vs baseline: 1.1129x; 1.0110x over previous
import functools

import jax
import jax.numpy as jnp
from jax import lax
from jax.experimental import pallas as pl
from jax.experimental.pallas import tpu as pltpu
from jax.experimental.pallas import tpu_sc as plsc

EPS = 1e-12


@functools.cache
def _make_sc_gather(nb, seq, V, D):
    info = plsc.get_sparse_core_info()
    NC, NS = info.num_cores, info.num_subcores
    NW = NC * NS
    B = nb * seq
    assert B % NW == 0
    b_per_w = B // NW
    assert seq % b_per_w == 0
    CH = min(128, b_per_w)
    assert b_per_w % CH == 0
    n_ch = b_per_w // CH
    mesh = plsc.VectorSubcoreMesh(core_axis_name="c", subcore_axis_name="s")

    @functools.partial(
        pl.kernel,
        mesh=mesh,
        out_type=jax.ShapeDtypeStruct((B, D), jnp.float32),
        scratch_types=[
            pltpu.VMEM((b_per_w,), jnp.int32),
            pltpu.VMEM((b_per_w, D), jnp.float32),
        ]
        + [pltpu.SemaphoreType.DMA] * (2 * n_ch),
    )
    def gather(idx_hbm, table_hbm, out_hbm, idx_v, rows_v, *sems):
        wid = lax.axis_index("s") * NC + lax.axis_index("c")
        base = wid * b_per_w
        row = base // seq
        col = base % seq
        pltpu.sync_copy(idx_hbm.at[row, pl.ds(col, b_per_w)], idx_v)
        gathers = [
            pltpu.async_copy(
                table_hbm.at[idx_v.at[pl.ds(j * CH, CH)]],
                rows_v.at[pl.ds(j * CH, CH)],
                sems[j],
            )
            for j in range(n_ch)
        ]
        writes = []
        for j in range(n_ch):
            gathers[j].wait()
            writes.append(
                pltpu.async_copy(
                    rows_v.at[pl.ds(j * CH, CH)],
                    out_hbm.at[pl.ds(base + j * CH, CH)],
                    sems[n_ch + j],
                )
            )
        for w in writes:
            w.wait()

    return gather


@functools.cache
def _make_tc_proj(B, D, H, BT, NB):
    S = B // BT

    def body(x_hbm, g_ref, bt_ref, w_ref, b_ref, o_hbm, xb, ob, sx, so):
        def xcopy(s):
            return pltpu.make_async_copy(
                x_hbm.at[pl.ds(s * BT, BT)], xb.at[s % NB], sx.at[s % NB]
            )

        def ocopy(s):
            return pltpu.make_async_copy(
                ob.at[s % NB], o_hbm.at[pl.ds(s * BT, BT)], so.at[s % NB]
            )

        for s in range(min(NB, S)):
            xcopy(s).start()
        gam = g_ref[...][None, :]
        bet = bt_ref[...][None, :]
        w = w_ref[...]
        bias = b_ref[...][None, :]
        for s in range(S):
            xcopy(s).wait()
            x = xb[s % NB]
            mean = jnp.mean(x, axis=1, keepdims=True)
            xc = x - mean
            var = jnp.mean(xc * xc, axis=1, keepdims=True)
            xn = xc * lax.rsqrt(var + EPS) * gam + bet
            if s >= NB:
                ocopy(s - NB).wait()
            ob[s % NB] = (
                jnp.dot(xn, w, preferred_element_type=jnp.float32) + bias
            )
            ocopy(s).start()
            if s + NB < S:
                xcopy(s + NB).start()
        for s in range(max(S - NB, 0), S):
            ocopy(s).wait()

    return pl.pallas_call(
        body,
        in_specs=[
            pl.BlockSpec(memory_space=pl.ANY),
            pl.BlockSpec(memory_space=pltpu.MemorySpace.VMEM),
            pl.BlockSpec(memory_space=pltpu.MemorySpace.VMEM),
            pl.BlockSpec(memory_space=pltpu.MemorySpace.VMEM),
            pl.BlockSpec(memory_space=pltpu.MemorySpace.VMEM),
        ],
        out_specs=pl.BlockSpec(memory_space=pl.ANY),
        out_shape=jax.ShapeDtypeStruct((B, H), jnp.float32),
        scratch_shapes=[
            pltpu.VMEM((NB, BT, D), jnp.float32),
            pltpu.VMEM((NB, BT, H), jnp.float32),
            pltpu.SemaphoreType.DMA((NB,)),
            pltpu.SemaphoreType.DMA((NB,)),
        ],
    )


@jax.jit
def kernel(input_ids, table, gamma, beta, W, b):
    nb, seq = input_ids.shape
    V, D = table.shape
    H = W.shape[1]
    B = nb * seq
    rows = _make_sc_gather(nb, seq, V, D)(input_ids, table)
    out = _make_tc_proj(B, D, H, 512, 6)(rows, gamma, beta, W, b)
    return out.reshape(nb, seq, H)
